# baseline (speedup 1.0000x reference)
import functools

import jax
import jax.numpy as jnp
from jax import lax
from jax.experimental import pallas as pl
from jax.experimental.pallas import tpu as pltpu
from jax.experimental.pallas import tpu_sc as plsc

B, N, K = 64, 32768, 200
THRESHOLD = 0.05
L = 16
NCHUNKS = N // L
NGROUPS = NCHUNKS // 32
KCHUNKS = (K + L - 1) // L
INT_MIN = -(2**31)
IBIG = 2**31 - 1
K005 = 0x3D4CCCCD


def _splat(x, dtype=jnp.int32):
    return lax.broadcast(jnp.asarray(x, dtype), (L,))


def _zero_buf(buf, nchunks, zv):
    def zb(j, _):
        buf[pl.ds(j * L, L)] = zv
        return 0

    lax.fori_loop(0, nchunks, zb, 0, unroll=8)


def _select_level(hist, nchunks, kp):
    lane = lax.iota(jnp.int32, L)

    def body(j, carry):
        found, bstar, kpn, esel, total = carry
        jj = nchunks - 1 - j
        h = hist[pl.ds(jj * L, L)]
        suf = lax.rev(plsc.cumsum(lax.rev(h, (0,))), (0,))
        ge = suf + _splat(total)
        csum = jnp.max(suf)
        kpv = _splat(kp)
        ncnt = jnp.sum(jnp.where(ge >= kpv, jnp.int32(1), jnp.int32(0)))
        hit = jnp.logical_and(found == 0, ncnt > 0)
        lstar = ncnt - 1
        hsel = jnp.max(jnp.where(lane == _splat(lstar), h, jnp.int32(0)))
        gesel = jnp.max(jnp.where(lane == _splat(lstar), ge, jnp.int32(0)))
        nb = jj * L + lstar
        nk = kp - (gesel - hsel)
        return (
            jnp.where(hit, jnp.int32(1), found),
            jnp.where(hit, nb, bstar),
            jnp.where(hit, nk, kpn),
            jnp.where(hit, hsel, esel),
            total + csum,
        )

    found, bstar, kpn, esel, _ = lax.fori_loop(
        0, nchunks, body,
        (jnp.int32(0), jnp.int32(0), jnp.int32(0), jnp.int32(0), jnp.int32(0)),
    )
    return bstar, kpn, esel, found


def _sc_body(
    scores_hbm, vmask_hbm, out_hbm, bs_hbm, bi_hbm,
    sbuf, mbuf, cidx, dense, hist, bsb, bib
):
    info = plsc.get_sparse_core_info()
    nc = info.num_cores
    wid = lax.axis_index("s") * nc + lax.axis_index("c")
    lane = lax.iota(jnp.int32, L)
    onesv = jnp.ones((L,), jnp.int32)
    zerov = jnp.zeros((L,), jnp.int32)
    minv = _splat(INT_MIN)
    thv = _splat(THRESHOLD, jnp.float32)
    zf = jnp.zeros((L,), jnp.float32)
    kK = jnp.int32(K)

    _zero_buf(dense, NCHUNKS, zf)

    for rr in range(2):
        row = wid * 2 + rr
        pltpu.sync_copy(scores_hbm.at[row], sbuf)
        pltpu.sync_copy(vmask_hbm.at[row], mbuf)

        _zero_buf(hist, 64, zerov)

        def passA(g, carry):
            off, gidxv = carry
            tw = mbuf[pl.ds(g * L, L)]
            for j in range(32):
                base = (g * 32 + j) * L
                s = sbuf[pl.ds(base, L)]
                i = plsc.bitcast(s, jnp.int32)
                viab = lax.bitwise_and(
                    lax.shift_right_logical(tw, _splat(j)), onesv
                ) != zerov
                m = jnp.logical_and(viab, s >= thv)
                plsc.store_compressed(cidx.at[pl.ds(off, L)], gidxv, mask=m)
                off = off + plsc.all_reduce_population_count(m)[0]
                bin3 = lax.shift_right_logical(i, _splat(21))
                plsc.addupdate_scatter(hist, [bin3], onesv, mask=m)
                gidxv = gidxv + _splat(L)
            return off, gidxv

        noff, _ = lax.fori_loop(0, NGROUPS, passA, (jnp.int32(0), lane))
        noffv = _splat(noff)
        candc = lax.shift_right_logical(noff + jnp.int32(L - 1), jnp.int32(4))
        b3, k1, _, found = _select_level(hist, 64, kK)

        _zero_buf(hist, 128, zerov)
        b3v = _splat(b3)

        def passB(c, carry):
            kmaxv, idxv, posv = carry
            ci = cidx[pl.ds(c * L, L)]
            valid = posv < noffv
            k = plsc.bitcast(plsc.load_gather(sbuf, [ci], mask=valid), jnp.int32)
            hi10 = lax.shift_right_logical(k, _splat(21))
            bin2 = lax.bitwise_and(lax.shift_right_logical(k, _splat(10)), _splat(0x7FF))
            plsc.addupdate_scatter(
                hist, [bin2], onesv,
                mask=jnp.logical_and(valid, hi10 == b3v),
            )
            kv = jnp.where(valid, k, minv)
            gt = kv > kmaxv
            kmaxv = jnp.where(gt, kv, kmaxv)
            idxv = jnp.where(gt, ci, idxv)
            return kmaxv, idxv, posv + _splat(L)

        kmaxv, idxv, _ = lax.fori_loop(0, candc, passB, (minv, zerov, lane))
        mkey = jnp.max(kmaxv)
        bidx_cand = jnp.min(jnp.where(kmaxv == _splat(mkey), idxv, _splat(IBIG)))
        b2, k2, _, _ = _select_level(hist, 128, k1)

        _zero_buf(hist, 64, zerov)
        sel21v = _splat(jnp.left_shift(b3, jnp.int32(11)) | b2)

        def passC(c, posv):
            ci = cidx[pl.ds(c * L, L)]
            valid = posv < noffv
            k = plsc.bitcast(plsc.load_gather(sbuf, [ci], mask=valid), jnp.int32)
            p21 = lax.shift_right_logical(k, _splat(10))
            bin1 = lax.bitwise_and(k, _splat(0x3FF))
            plsc.addupdate_scatter(
                hist, [bin1], onesv,
                mask=jnp.logical_and(valid, p21 == sel21v),
            )
            return posv + _splat(L)

        lax.fori_loop(0, candc, passC, lane)
        b1, k3, ecnt, _ = _select_level(hist, 64, k2)

        tkey = (
            jnp.left_shift(b3, jnp.int32(21))
            | jnp.left_shift(b2, jnp.int32(10))
            | b1
        )
        tkey = jnp.where(found > 0, tkey, jnp.int32(K005))
        tv = _splat(tkey)
        rv = _splat(k3)

        def f_fast(_):
            def fb(c, carry):
                off2, posv = carry
                ci = cidx[pl.ds(c * L, L)]
                valid = posv < noffv
                k = plsc.bitcast(
                    plsc.load_gather(sbuf, [ci], mask=valid), jnp.int32
                )
                m = jnp.logical_and(valid, k >= tv)
                plsc.store_compressed(cidx.at[pl.ds(off2, L)], ci, mask=m)
                return (
                    off2 + plsc.all_reduce_population_count(m)[0],
                    posv + _splat(L),
                )

            off2, _ = lax.fori_loop(0, candc, fb, (jnp.int32(0), lane))
            return off2

        def f_ration(_):
            def fb(c, carry):
                off2, cntv, posv = carry
                ci = cidx[pl.ds(c * L, L)]
                valid = posv < noffv
                k = plsc.bitcast(
                    plsc.load_gather(sbuf, [ci], mask=valid), jnp.int32
                )
                eq = jnp.logical_and(valid, k == tv)
                eqi = jnp.where(eq, jnp.int32(1), jnp.int32(0))
                pexc = plsc.cumsum(eqi) - eqi
                m = jnp.logical_and(
                    valid,
                    jnp.logical_or(
                        k > tv, jnp.logical_and(eq, cntv + pexc < rv)
                    ),
                )
                plsc.store_compressed(cidx.at[pl.ds(off2, L)], ci, mask=m)
                return (
                    off2 + plsc.all_reduce_population_count(m)[0],
                    cntv + plsc.all_reduce_population_count(eq),
                    posv + _splat(L),
                )

            off2, _, _ = lax.fori_loop(
                0, candc, fb, (jnp.int32(0), _splat(0), lane)
            )
            return off2

        nkept = lax.cond(ecnt > k3, f_ration, f_fast, 0)
        nkv = _splat(nkept)

        kept = []
        for t in range(KCHUNKS):
            ci = cidx[pl.ds(t * L, L)]
            sel = _splat(t * L) + lane < nkv
            v = plsc.load_gather(sbuf, [ci], mask=sel)
            plsc.store_scatter(dense, [ci], v, mask=sel)
            kept.append((ci, sel))
        pltpu.sync_copy(dense, out_hbm.at[row])
        for ci, sel in kept:
            plsc.store_scatter(dense, [ci], zf, mask=sel)

        def arg_fallback(_):
            ninf = _splat(float("-inf"), jnp.float32)

            def abody(g, carry):
                fmaxv, idxv2, gidxv = carry
                tw = mbuf[pl.ds(g * L, L)]
                for j in range(32):
                    base = (g * 32 + j) * L
                    s = sbuf[pl.ds(base, L)]
                    viab = lax.bitwise_and(
                        lax.shift_right_logical(tw, _splat(j)), onesv
                    ) != zerov
                    fm = jnp.where(viab, s, ninf)
                    gt = fm > fmaxv
                    fmaxv = jnp.where(gt, fm, fmaxv)
                    idxv2 = jnp.where(gt, gidxv, idxv2)
                    gidxv = gidxv + _splat(L)
                return fmaxv, idxv2, gidxv

            fmaxv, idxv2, _ = lax.fori_loop(
                0, NGROUPS, abody, (ninf, zerov, lane)
            )
            mf = jnp.max(fmaxv)
            return jnp.min(
                jnp.where(fmaxv == _splat(mf, jnp.float32), idxv2, _splat(IBIG))
            )

        bidx = lax.cond(noff > 0, lambda _: bidx_cand, arg_fallback, 0)
        bidx = jnp.where(bidx == IBIG, jnp.int32(0), bidx)

        bsb[...] = plsc.bitcast(
            _splat(jnp.where(noff > 0, mkey, jnp.int32(0))), jnp.float32
        )
        bib[...] = _splat(bidx)
        pltpu.sync_copy(bsb, bs_hbm.at[row])
        pltpu.sync_copy(bib, bi_hbm.at[row])


@jax.jit
def _run(scores, vmaskT):
    mesh = plsc.VectorSubcoreMesh(core_axis_name="c", subcore_axis_name="s")
    fn = pl.kernel(
        _sc_body,
        out_type=[
            jax.ShapeDtypeStruct((B, N), jnp.float32),
            jax.ShapeDtypeStruct((B, L), jnp.float32),
            jax.ShapeDtypeStruct((B, L), jnp.int32),
        ],
        mesh=mesh,
        compiler_params=pltpu.CompilerParams(needs_layout_passes=False),
        scratch_types=[
            pltpu.VMEM((N,), jnp.float32),
            pltpu.VMEM((NGROUPS * L,), jnp.int32),
            pltpu.VMEM((N + L,), jnp.int32),
            pltpu.VMEM((N,), jnp.float32),
            pltpu.VMEM((2048,), jnp.int32),
            pltpu.VMEM((L,), jnp.float32),
            pltpu.VMEM((L,), jnp.int32),
        ],
    )
    return fn(scores, vmaskT)


def kernel(scores, viable, k):
    v4 = viable.reshape(B, NGROUPS, 32, L).astype(jnp.uint32)
    sh = jnp.arange(32, dtype=jnp.uint32)[None, None, :, None]
    vmaskT = lax.bitcast_convert_type(
        jnp.sum(v4 << sh, axis=2), jnp.int32
    ).reshape(B, NGROUPS * L)
    out, bs, bi = _run(scores, vmaskT)
    return out, bs[:, 0], bi[:, 0]
